# Initial kernel scaffold; baseline (speedup 1.0000x reference)
import jax
import jax.numpy as jnp
from jax.experimental import pallas as pl


def kernel(node_feat, node_att, edge_feat, edge_att, edge_index, W_node, b_node, W_apply_node, b_apply_node):
    raise NotImplementedError("write your pallas kernel here")


# speedup vs baseline: 7.5269x; 7.5269x over previous
import functools

import jax
import jax.numpy as jnp
from jax import lax
from jax.experimental import pallas as pl
from jax.experimental.pallas import tpu as pltpu
from jax.experimental.pallas import tpu_sc as plsc

_NC, _NS, _L = 2, 16, 16
_GSZ = 128


def _node_fc_body(x_ref, att_ref, w_ref, b_ref, feat_ref, afeat_ref):
    f = jnp.dot(x_ref[...], w_ref[...], preferred_element_type=jnp.float32)
    f = f + b_ref[...]
    feat_ref[...] = f
    afeat_ref[...] = att_ref[...] * f


def _node_fc(x, att2, w, b2):
    n, d = x.shape
    blk = 1000
    return pl.pallas_call(
        _node_fc_body,
        grid=(n // blk,),
        in_specs=[
            pl.BlockSpec((blk, d), lambda i: (i, 0)),
            pl.BlockSpec((blk, 1), lambda i: (i, 0)),
            pl.BlockSpec((d, d), lambda i: (0, 0)),
            pl.BlockSpec((1, d), lambda i: (0, 0)),
        ],
        out_specs=[pl.BlockSpec((blk, d), lambda i: (i, 0))] * 2,
        out_shape=[jax.ShapeDtypeStruct((n, d), jnp.float32)] * 2,
    )(x, att2, w, b2)


def _apply_body(a1_ref, a2_ref, feat_ref, w_ref, b_ref, att_ref, out_ref):
    w = w_ref[...]
    d = w.shape[1]
    h = jnp.dot(a1_ref[...], w[0:d], preferred_element_type=jnp.float32)
    h = h + jnp.dot(a2_ref[...], w[d:2 * d], preferred_element_type=jnp.float32)
    h = h + jnp.dot(feat_ref[...], w[2 * d:3 * d], preferred_element_type=jnp.float32)
    h = h + b_ref[...]
    out_ref[...] = att_ref[...] * jnp.maximum(h, 0.0)


def _apply_node(a1, a2, feat, w, b2, att2):
    n, d = feat.shape
    blk = 1000
    return pl.pallas_call(
        _apply_body,
        grid=(n // blk,),
        in_specs=[
            pl.BlockSpec((blk, d), lambda i: (i, 0)),
            pl.BlockSpec((blk, d), lambda i: (i, 0)),
            pl.BlockSpec((blk, d), lambda i: (i, 0)),
            pl.BlockSpec((3 * d, d), lambda i: (0, 0)),
            pl.BlockSpec((1, d), lambda i: (0, 0)),
            pl.BlockSpec((blk, 1), lambda i: (i, 0)),
        ],
        out_specs=pl.BlockSpec((blk, d), lambda i: (i, 0)),
        out_shape=jax.ShapeDtypeStruct((n, d), jnp.float32),
    )(a1, a2, feat, w, b2, att2)


@functools.cache
def _sc_scatter(n, e, d):
    g_total = e // _GSZ
    q, r = divmod(g_total, _NS)
    rpt = (n // _NS) // 8 * 8
    tail = n - _NS * rpt
    mesh = plsc.VectorSubcoreMesh(
        core_axis_name="c", subcore_axis_name="s",
        num_cores=_NC, num_subcores=_NS)

    @functools.partial(
        pl.kernel,
        out_type=(jax.ShapeDtypeStruct((n, d), jnp.float32),
                  jax.ShapeDtypeStruct((n, d), jnp.float32)),
        mesh=mesh,
        scratch_types=[
            pltpu.VMEM_SHARED((n, d), jnp.float32),
            pltpu.VMEM((_GSZ,), jnp.int32),
            pltpu.VMEM((_GSZ,), jnp.int32),
            pltpu.VMEM((_GSZ,), jnp.float32),
            pltpu.VMEM((_GSZ, d), jnp.float32),
            pltpu.SemaphoreType.DMA,
        ],
    )
    def sc_fn(afeat_hbm, ef_hbm, src_hbm, dst_hbm, eatt_hbm, zeros_hbm,
              agg1_hbm, agg2_hbm,
              acc, sidx_v, didx_v, att_v, rows_v, sem):
        cid = lax.axis_index("c")
        sid = lax.axis_index("s")

        zb = sid * rpt
        pltpu.sync_copy(zeros_hbm.at[pl.ds(zb, rpt)],
                        acc.at[pl.ds(zb, rpt)])
        if tail:
            @pl.when(sid == 0)
            def _():
                pltpu.sync_copy(zeros_hbm.at[pl.ds(_NS * rpt, tail)],
                                acc.at[pl.ds(_NS * rpt, tail)])
        plsc.subcore_barrier()

        start = sid * q + jnp.minimum(sid, r)
        count = q + jnp.where(sid < r, 1, 0)

        @pl.when(cid == 0)
        def _():
            def body(i, carry):
                base = (start + i) * _GSZ
                pltpu.sync_copy(src_hbm.at[pl.ds(base, _GSZ)], sidx_v)
                pltpu.sync_copy(dst_hbm.at[pl.ds(base, _GSZ)], didx_v)
                pltpu.async_copy(afeat_hbm.at[sidx_v], rows_v, sem).wait()
                pltpu.sync_copy(rows_v, acc.at[didx_v], add=True)
                return carry
            lax.fori_loop(0, count, body, 0)

        @pl.when(cid == 1)
        def _():
            def body(i, carry):
                base = (start + i) * _GSZ
                pltpu.sync_copy(dst_hbm.at[pl.ds(base, _GSZ)], didx_v)
                pltpu.sync_copy(eatt_hbm.at[pl.ds(base, _GSZ)], att_v)
                pltpu.async_copy(ef_hbm.at[pl.ds(base, _GSZ)], rows_v, sem).wait()

                def ebody(blk, c2):
                    att16 = att_v[pl.ds(blk * _L, _L)]
                    for le in range(_L):
                        s = att16[le]
                        j = blk * _L + le
                        for k in range(d // _L):
                            rv = rows_v[j, pl.ds(k * _L, _L)]
                            rows_v[j, pl.ds(k * _L, _L)] = rv * s
                    return c2
                lax.fori_loop(0, _GSZ // _L, ebody, 0)
                pltpu.sync_copy(rows_v, acc.at[didx_v], add=True)
                return carry
            lax.fori_loop(0, count, body, 0)

        plsc.subcore_barrier()

        db = sid * rpt

        @pl.when(cid == 0)
        def _():
            pltpu.sync_copy(acc.at[pl.ds(db, rpt)],
                            agg1_hbm.at[pl.ds(db, rpt)])
            if tail:
                @pl.when(sid == 0)
                def _():
                    pltpu.sync_copy(acc.at[pl.ds(_NS * rpt, tail)],
                                    agg1_hbm.at[pl.ds(_NS * rpt, tail)])

        @pl.when(cid == 1)
        def _():
            pltpu.sync_copy(acc.at[pl.ds(db, rpt)],
                            agg2_hbm.at[pl.ds(db, rpt)])
            if tail:
                @pl.when(sid == 0)
                def _():
                    pltpu.sync_copy(acc.at[pl.ds(_NS * rpt, tail)],
                                    agg2_hbm.at[pl.ds(_NS * rpt, tail)])

    return sc_fn


def kernel(node_feat, node_att, edge_feat, edge_att, edge_index,
           W_node, b_node, W_apply_node, b_apply_node):
    n, d = node_feat.shape
    e = edge_feat.shape[0]
    att2 = node_att[:, None]
    b2 = b_node[None, :]
    ba2 = b_apply_node[None, :]

    feat, a_feat = _node_fc(node_feat, att2, W_node, b2)

    src = edge_index[0]
    dst = edge_index[1]
    zeros = jnp.zeros((n, d), jnp.float32)
    agg1, agg2 = _sc_scatter(n, e, d)(
        a_feat, edge_feat, src, dst, edge_att, zeros)

    return _apply_node(agg1, agg2, feat, W_apply_node, ba2, att2)


# speedup vs baseline: 15.0557x; 2.0003x over previous
import functools

import jax
import jax.numpy as jnp
from jax import lax
from jax.experimental import pallas as pl
from jax.experimental.pallas import tpu as pltpu
from jax.experimental.pallas import tpu_sc as plsc

_NC, _NS, _L = 2, 16, 16
_GSZ = 128


def _node_fc_body(x_ref, att_ref, w_ref, b_ref, feat_ref, afeat_ref):
    f = jnp.dot(x_ref[...], w_ref[...], preferred_element_type=jnp.float32)
    f = f + b_ref[...]
    feat_ref[...] = f
    afeat_ref[...] = att_ref[...] * f


def _node_fc(x, att2, w, b2):
    n, d = x.shape
    blk = 1000
    return pl.pallas_call(
        _node_fc_body,
        grid=(n // blk,),
        in_specs=[
            pl.BlockSpec((blk, d), lambda i: (i, 0)),
            pl.BlockSpec((blk, 1), lambda i: (i, 0)),
            pl.BlockSpec((d, d), lambda i: (0, 0)),
            pl.BlockSpec((1, d), lambda i: (0, 0)),
        ],
        out_specs=[pl.BlockSpec((blk, d), lambda i: (i, 0))] * 2,
        out_shape=[jax.ShapeDtypeStruct((n, d), jnp.float32)] * 2,
    )(x, att2, w, b2)


def _apply_body(a1_ref, a2_ref, feat_ref, w_ref, b_ref, att_ref, out_ref):
    w = w_ref[...]
    d = w.shape[1]
    h = jnp.dot(a1_ref[...], w[0:d], preferred_element_type=jnp.float32)
    h = h + jnp.dot(a2_ref[...], w[d:2 * d], preferred_element_type=jnp.float32)
    h = h + jnp.dot(feat_ref[...], w[2 * d:3 * d], preferred_element_type=jnp.float32)
    h = h + b_ref[...]
    out_ref[...] = att_ref[...] * jnp.maximum(h, 0.0)


def _apply_node(a1, a2, feat, w, b2, att2):
    n, d = feat.shape
    blk = 1000
    return pl.pallas_call(
        _apply_body,
        grid=(n // blk,),
        in_specs=[
            pl.BlockSpec((blk, d), lambda i: (i, 0)),
            pl.BlockSpec((blk, d), lambda i: (i, 0)),
            pl.BlockSpec((blk, d), lambda i: (i, 0)),
            pl.BlockSpec((3 * d, d), lambda i: (0, 0)),
            pl.BlockSpec((1, d), lambda i: (0, 0)),
            pl.BlockSpec((blk, 1), lambda i: (i, 0)),
        ],
        out_specs=pl.BlockSpec((blk, d), lambda i: (i, 0)),
        out_shape=jax.ShapeDtypeStruct((n, d), jnp.float32),
    )(a1, a2, feat, w, b2, att2)


_NBUF = 2
_SB = 8
_TGRP = 160


@functools.cache
def _sc_scatter(n, e, d):
    g_total = e // _GSZ
    assert _NS * _TGRP >= g_total
    assert g_total % 2 == 0 and _TGRP % 2 == 0
    rpt = (n // _NS) // 8 * 8
    tail = n - _NS * rpt
    mesh = plsc.VectorSubcoreMesh(
        core_axis_name="c", subcore_axis_name="s",
        num_cores=_NC, num_subcores=_NS)

    @functools.partial(
        pl.kernel,
        out_type=(jax.ShapeDtypeStruct((n, d), jnp.float32),
                  jax.ShapeDtypeStruct((n, d), jnp.float32)),
        mesh=mesh,
        scratch_types=[
            pltpu.VMEM_SHARED((n, d), jnp.float32),
            pltpu.VMEM((2, _SB, _GSZ), jnp.int32),
            pltpu.VMEM((2, _SB, _GSZ), jnp.int32),
            pltpu.VMEM((2, _SB, _GSZ), jnp.float32),
            pltpu.VMEM((_NBUF, _GSZ, d), jnp.float32),
            [pltpu.SemaphoreType.DMA] * _NBUF,
            [pltpu.SemaphoreType.DMA] * _NBUF,
            [pltpu.SemaphoreType.DMA] * 2,
        ],
    )
    def sc_fn(afeat_hbm, ef_hbm, src_hbm, dst_hbm, eatt_hbm, zeros_hbm,
              agg1_hbm, agg2_hbm,
              acc, didx_v, sidx_v, att_v, rows_v, gsems, ssems, isems):
        cid = lax.axis_index("c")
        sid = lax.axis_index("s")

        zb = sid * rpt
        pltpu.sync_copy(zeros_hbm.at[pl.ds(zb, rpt)],
                        acc.at[pl.ds(zb, rpt)])
        if tail:
            @pl.when(sid == 0)
            def _():
                pltpu.sync_copy(zeros_hbm.at[pl.ds(_NS * rpt, tail)],
                                acc.at[pl.ds(_NS * rpt, tail)])

        start = sid * _TGRP
        count = jnp.clip(g_total - start, 0, _TGRP)
        nsb = (count + _SB - 1) // _SB

        def fire_gather(li, idx_row, b):
            @pl.when(cid == 0)
            def _():
                pltpu.async_copy(afeat_hbm.at[idx_row], rows_v.at[b],
                                 gsems[b])

            @pl.when(cid == 1)
            def _():
                pltpu.async_copy(
                    ef_hbm.at[pl.ds((start + li) * _GSZ, _GSZ)],
                    rows_v.at[b], gsems[b])

        def wait_gather(b):
            pltpu.make_async_copy(ef_hbm.at[pl.ds(0, _GSZ)],
                                  rows_v.at[b], gsems[b]).wait()

        def wait_scatter(b):
            pltpu.make_async_copy(rows_v.at[b], acc.at[didx_v.at[0, 0]],
                                  ssems[b]).wait()

        def load_idx_block(dst_buf, row0, sync):
            if sync:
                pltpu.sync_copy(dst_hbm.at[pl.ds(row0, _SB)],
                                didx_v.at[dst_buf])
            else:
                pltpu.async_copy(dst_hbm.at[pl.ds(row0, _SB)],
                                 didx_v.at[dst_buf], isems[0])

            @pl.when(cid == 0)
            def _():
                if sync:
                    pltpu.sync_copy(src_hbm.at[pl.ds(row0, _SB)],
                                    sidx_v.at[dst_buf])
                else:
                    pltpu.async_copy(src_hbm.at[pl.ds(row0, _SB)],
                                     sidx_v.at[dst_buf], isems[1])

            @pl.when(cid == 1)
            def _():
                if sync:
                    pltpu.sync_copy(eatt_hbm.at[pl.ds(row0, _SB)],
                                    att_v.at[dst_buf])
                else:
                    pltpu.async_copy(eatt_hbm.at[pl.ds(row0, _SB)],
                                     att_v.at[dst_buf], isems[1])

        load_idx_block(0, start, sync=True)

        @pl.when(0 < count)
        def _():
            fire_gather(0, sidx_v.at[0, 0], 0)

        plsc.subcore_barrier()

        def superblock(sb, carry):
            ib = lax.rem(sb, 2)
            nb = 1 - ib
            have_next = sb + 1 < nsb

            @pl.when(have_next)
            def _():
                load_idx_block(nb, start + (sb + 1) * _SB, sync=False)

            for j in range(_SB):
                li = sb * _SB + j
                b = j % 2
                bn = (j + 1) % 2
                lnext = li + 1

                if j < _SB - 1:
                    @pl.when(lnext < count)
                    def _(j=j, bn=bn, lnext=lnext):
                        @pl.when(lnext >= 2)
                        def _():
                            wait_scatter(bn)
                        fire_gather(lnext, sidx_v.at[ib, j + 1], bn)
                else:
                    @pl.when(have_next)
                    def _(bn=bn, lnext=lnext):
                        pltpu.make_async_copy(
                            dst_hbm.at[pl.ds(start, _SB)],
                            didx_v.at[nb], isems[0]).wait()
                        pltpu.make_async_copy(
                            dst_hbm.at[pl.ds(start, _SB)],
                            sidx_v.at[nb], isems[1]).wait()
                        wait_scatter(bn)
                        fire_gather(lnext, sidx_v.at[nb, 0], bn)

                @pl.when(li < count)
                def _(j=j, b=b, li=li):
                    wait_gather(b)

                    @pl.when(cid == 1)
                    def _():
                        def ebody(blk16, c2):
                            att16 = att_v[ib, j, pl.ds(blk16 * _L, _L)]
                            for le in range(_L):
                                s = att16[le]
                                jr = blk16 * _L + le
                                for k in range(d // _L):
                                    rv = rows_v[b, jr, pl.ds(k * _L, _L)]
                                    rows_v[b, jr, pl.ds(k * _L, _L)] = rv * s
                            return c2
                        lax.fori_loop(0, _GSZ // _L, ebody, 0)

                    pltpu.async_copy(rows_v.at[b], acc.at[didx_v.at[ib, j]],
                                     ssems[b], add=True)

            return carry

        lax.fori_loop(0, nsb, superblock, 0)

        @pl.when(count >= 2)
        def _():
            wait_scatter(0)

        @pl.when(count >= 1)
        def _():
            wait_scatter(1)

        plsc.subcore_barrier()

        db = sid * rpt

        @pl.when(cid == 0)
        def _():
            pltpu.sync_copy(acc.at[pl.ds(db, rpt)],
                            agg1_hbm.at[pl.ds(db, rpt)])
            if tail:
                @pl.when(sid == 0)
                def _():
                    pltpu.sync_copy(acc.at[pl.ds(_NS * rpt, tail)],
                                    agg1_hbm.at[pl.ds(_NS * rpt, tail)])

        @pl.when(cid == 1)
        def _():
            pltpu.sync_copy(acc.at[pl.ds(db, rpt)],
                            agg2_hbm.at[pl.ds(db, rpt)])
            if tail:
                @pl.when(sid == 0)
                def _():
                    pltpu.sync_copy(acc.at[pl.ds(_NS * rpt, tail)],
                                    agg2_hbm.at[pl.ds(_NS * rpt, tail)])

    return sc_fn


def kernel(node_feat, node_att, edge_feat, edge_att, edge_index,
           W_node, b_node, W_apply_node, b_apply_node):
    n, d = node_feat.shape
    e = edge_feat.shape[0]
    att2 = node_att[:, None]
    b2 = b_node[None, :]
    ba2 = b_apply_node[None, :]

    feat, a_feat = _node_fc(node_feat, att2, W_node, b2)

    g_total = e // _GSZ
    pad_g = _NS * _TGRP - g_total
    src2 = jnp.pad(edge_index[0].reshape(g_total, _GSZ), ((0, pad_g), (0, 0)))
    dst2 = jnp.pad(edge_index[1].reshape(g_total, _GSZ), ((0, pad_g), (0, 0)))
    eatt2 = jnp.pad(edge_att.reshape(g_total, _GSZ), ((0, pad_g), (0, 0)))
    zeros = jnp.zeros((n, d), jnp.float32)
    agg1, agg2 = _sc_scatter(n, e, d)(
        a_feat, edge_feat, src2, dst2, eatt2, zeros)

    return _apply_node(agg1, agg2, feat, W_apply_node, ba2, att2)
